# Initial kernel scaffold; baseline (speedup 1.0000x reference)
import jax
import jax.numpy as jnp
from jax.experimental import pallas as pl


def kernel(Z, rec_Z, w1, b1, w2, b2):
    raise NotImplementedError("write your pallas kernel here")


# speedup vs baseline: 1.0336x; 1.0336x over previous
import functools

import jax
import jax.numpy as jnp
from jax.experimental import pallas as pl
from jax.experimental.pallas import tpu as pltpu


def _round_up(x: int, m: int) -> int:
    return ((x + m - 1) // m) * m


def _disc_kernel(z_ref, rz_ref, w1a_ref, w1b_ref, b1_ref, w2_ref, b2_ref,
                 logits_ref, mid_ref, *, negative_slope):
    z = z_ref[...].astype(jnp.bfloat16)
    rz = rz_ref[...].astype(jnp.bfloat16)
    h = (jnp.dot(z, w1a_ref[...], preferred_element_type=jnp.float32)
         + jnp.dot(rz, w1b_ref[...], preferred_element_type=jnp.float32)
         + b1_ref[...])

    mid = jnp.where(h >= 0.0, h, negative_slope * h)
    mid_ref[...] = mid

    logits = (jnp.dot(mid.astype(jnp.bfloat16), w2_ref[...],
                      preferred_element_type=jnp.float32)
              + b2_ref[...])
    logits_ref[...] = logits


def kernel(Z, rec_Z, w1, b1, w2, b2):
    B, in_features = Z.shape
    out_features = w1.shape[1]
    n_classes = w2.shape[1]

    OUT_PAD = _round_up(out_features, 128)
    NC_PAD = _round_up(n_classes, 128)

    w1p = jnp.pad(w1, ((0, 0), (0, OUT_PAD - out_features))).astype(jnp.bfloat16)
    b1p = jnp.pad(b1, ((0, 0), (0, OUT_PAD - out_features)))
    w2p = jnp.pad(w2, ((0, OUT_PAD - out_features),
                       (0, NC_PAD - n_classes))).astype(jnp.bfloat16)
    b2p = jnp.pad(b2, ((0, 0), (0, NC_PAD - n_classes)))

    w1a = w1p[:in_features, :]
    w1b = w1p[in_features:, :]

    VMEM_BUDGET = 100 * 1024 * 1024
    tile_b = min(1024, _round_up(B, 8))

    def _tile_bytes(tb):
        per_row = (2 * in_features + OUT_PAD + NC_PAD) * 4
        weights = (2 * in_features * OUT_PAD + OUT_PAD * NC_PAD) * 2 \
            + (OUT_PAD + NC_PAD) * 4
        return 2 * tb * per_row + 2 * weights
    while tile_b > 8 and _tile_bytes(tile_b) > VMEM_BUDGET:
        tile_b //= 2
    tile_b = max(tile_b, 8)

    B_pad = _round_up(B, tile_b)
    if B_pad != B:
        Z_in = jnp.pad(Z, ((0, B_pad - B), (0, 0)))
        R_in = jnp.pad(rec_Z, ((0, B_pad - B), (0, 0)))
    else:
        Z_in, R_in = Z, rec_Z

    grid = (B_pad // tile_b,)

    body = functools.partial(_disc_kernel, negative_slope=0.2)

    flops = 2 * B_pad * (2 * in_features * OUT_PAD + OUT_PAD * NC_PAD)
    bytes_accessed = (
        4 * 2 * B_pad * in_features
        + 2 * (2 * in_features * OUT_PAD + OUT_PAD * NC_PAD)
        + 4 * (OUT_PAD + NC_PAD)
        + 4 * B_pad * (OUT_PAD + NC_PAD))

    logits_p, mid_p = pl.pallas_call(
        body,
        out_shape=(
            jax.ShapeDtypeStruct((B_pad, NC_PAD), jnp.float32),
            jax.ShapeDtypeStruct((B_pad, OUT_PAD), jnp.float32),
        ),
        grid=grid,
        in_specs=[
            pl.BlockSpec((tile_b, in_features), lambda i: (i, 0)),
            pl.BlockSpec((tile_b, in_features), lambda i: (i, 0)),
            pl.BlockSpec((in_features, OUT_PAD), lambda i: (0, 0)),
            pl.BlockSpec((in_features, OUT_PAD), lambda i: (0, 0)),
            pl.BlockSpec((1, OUT_PAD), lambda i: (0, 0)),
            pl.BlockSpec((OUT_PAD, NC_PAD), lambda i: (0, 0)),
            pl.BlockSpec((1, NC_PAD), lambda i: (0, 0)),
        ],
        out_specs=(
            pl.BlockSpec((tile_b, NC_PAD), lambda i: (i, 0)),
            pl.BlockSpec((tile_b, OUT_PAD), lambda i: (i, 0)),
        ),
        compiler_params=pltpu.CompilerParams(
            dimension_semantics=("parallel",),
            vmem_limit_bytes=VMEM_BUDGET,
        ),
        cost_estimate=pl.CostEstimate(
            flops=flops, transcendentals=0, bytes_accessed=bytes_accessed),
    )(Z_in, R_in, w1a, w1b, b1p, w2p, b2p)

    logits = logits_p[:B, :n_classes]
    mid = mid_p[:B, :out_features]
    return logits, mid
